# Initial kernel scaffold; baseline (speedup 1.0000x reference)
import jax
import jax.numpy as jnp
from jax.experimental import pallas as pl


def kernel(Q, K, V, mask):
    raise NotImplementedError("write your pallas kernel here")


# speedup vs baseline: 1.3563x; 1.3563x over previous
import math

import jax
import jax.numpy as jnp
from jax.experimental import pallas as pl
from jax.experimental.pallas import tpu as pltpu

HASH_LEN = 9
BQ = 512

_PS0 = 1.6666586697e-01
_PS1 = -4.2743422091e-02
_PS2 = -8.6563630030e-03
_QS1 = -7.0662963390e-01


def _acos(d):
    ax = jnp.abs(d)
    small = ax < 0.5
    z = jnp.where(small, d * d, 0.5 * (1.0 - ax))
    r = (z * (_PS0 + z * (_PS1 + z * _PS2))) / (1.0 + z * _QS1)
    s = jnp.sqrt(z)
    big_pos = 2.0 * (s + s * r)
    big = jnp.where(d > 0, big_pos, math.pi - big_pos)
    return jnp.where(small, math.pi / 2 - (d + d * r), big)


def _yoso_block(q_ref, k_ref, v_ref, mcol_ref, mrow_ref, o_ref):
    q = q_ref[0]
    k = k_ref[0]
    v = v_ref[0]
    dot = jax.lax.dot_general(
        q, k, (((1,), (1,)), ((), ())), preferred_element_type=jnp.float32)
    dot = jnp.clip(dot, -1.0, 1.0)
    p = 1.0 - _acos(dot) * (1.0 / math.pi)
    p2 = p * p
    p4 = p2 * p2
    p8 = p4 * p4
    p9 = p8 * p
    p9 = p9 * mcol_ref[0]
    x = jax.lax.dot_general(
        p9, v, (((1,), (0,)), ((), ())), preferred_element_type=jnp.float32)
    x = x * mrow_ref[0]
    norm = jnp.sqrt(jnp.sum(x * x, axis=-1, keepdims=True))
    o_ref[0] = x / (norm + 1e-6)


def kernel(Q, K, V, mask):
    B, H, S, D = Q.shape
    BH = B * H
    Qf = Q.reshape(BH, S, D)
    Kf = K.reshape(BH, S, D)
    Vf = V.reshape(BH, S, D)
    mask = mask.astype(Q.dtype)
    mcol = mask.reshape(B, 1, S)
    mrow = mask.reshape(B, S, 1)

    grid = (BH, S // BQ)
    out = pl.pallas_call(
        _yoso_block,
        grid=grid,
        in_specs=[
            pl.BlockSpec((1, BQ, D), lambda h, i: (h, i, 0)),
            pl.BlockSpec((1, S, D), lambda h, i: (h, 0, 0)),
            pl.BlockSpec((1, S, D), lambda h, i: (h, 0, 0)),
            pl.BlockSpec((1, 1, S), lambda h, i: (h // H, 0, 0)),
            pl.BlockSpec((1, BQ, 1), lambda h, i: (h // H, i, 0)),
        ],
        out_specs=pl.BlockSpec((1, BQ, D), lambda h, i: (h, i, 0)),
        out_shape=jax.ShapeDtypeStruct((BH, S, D), Q.dtype),
        compiler_params=pltpu.CompilerParams(
            dimension_semantics=("arbitrary", "arbitrary"),
        ),
    )(Qf, Kf, Vf, mcol, mrow)
    return out.reshape(B, H, S, D)


# speedup vs baseline: 1.8419x; 1.3580x over previous
import math

import jax
import jax.numpy as jnp
from jax.experimental import pallas as pl
from jax.experimental.pallas import tpu as pltpu

HASH_LEN = 9
BQ = 512

_G_COEFFS = (
    0.6366205811500549,
    0.1060168594121933,
    0.049960680305957794,
    0.0070031434297561646,
    0.11223539710044861,
    -0.17107734084129333,
    0.15588559210300446,
)


def _p9(dot):
    z = jnp.maximum(0.5 - 0.5 * dot, 0.0)
    s = z * jax.lax.rsqrt(z + 1e-30)
    g = _G_COEFFS[-1]
    for c in _G_COEFFS[-2::-1]:
        g = c + z * g
    p = 1.0 - s * g
    p2 = p * p
    p4 = p2 * p2
    p8 = p4 * p4
    return p8 * p


def _yoso_block(q_ref, k_ref, v_ref, mcol_ref, mrow_ref, o_ref):
    q = q_ref[0]
    k = k_ref[0]
    v = v_ref[0]
    dot = jax.lax.dot_general(
        q, k, (((1,), (1,)), ((), ())), preferred_element_type=jnp.float32)
    p9 = _p9(dot)
    p9 = p9 * mcol_ref[0]
    x = jax.lax.dot_general(
        p9, v, (((1,), (0,)), ((), ())), preferred_element_type=jnp.float32)
    x = x * mrow_ref[0]
    norm = jnp.sqrt(jnp.sum(x * x, axis=-1, keepdims=True))
    o_ref[0] = x / (norm + 1e-6)


def kernel(Q, K, V, mask):
    B, H, S, D = Q.shape
    BH = B * H
    Qf = Q.reshape(BH, S, D)
    Kf = K.reshape(BH, S, D)
    Vf = V.reshape(BH, S, D)
    mask = mask.astype(Q.dtype)
    mcol = mask.reshape(B, 1, S)
    mrow = mask.reshape(B, S, 1)

    grid = (BH, S // BQ)
    out = pl.pallas_call(
        _yoso_block,
        grid=grid,
        in_specs=[
            pl.BlockSpec((1, BQ, D), lambda h, i: (h, i, 0)),
            pl.BlockSpec((1, S, D), lambda h, i: (h, 0, 0)),
            pl.BlockSpec((1, S, D), lambda h, i: (h, 0, 0)),
            pl.BlockSpec((1, 1, S), lambda h, i: (h // H, 0, 0)),
            pl.BlockSpec((1, BQ, 1), lambda h, i: (h // H, i, 0)),
        ],
        out_specs=pl.BlockSpec((1, BQ, D), lambda h, i: (h, i, 0)),
        out_shape=jax.ShapeDtypeStruct((BH, S, D), Q.dtype),
        compiler_params=pltpu.CompilerParams(
            dimension_semantics=("arbitrary", "arbitrary"),
        ),
    )(Qf, Kf, Vf, mcol, mrow)
    return out.reshape(B, H, S, D)


# speedup vs baseline: 2.2122x; 1.2011x over previous
import math

import jax
import jax.numpy as jnp
from jax.experimental import pallas as pl
from jax.experimental.pallas import tpu as pltpu

HASH_LEN = 9
BQ = 1024

_G_COEFFS = (
    0.6366177201271057,
    0.1062883660197258,
    0.04400774464011192,
    0.05543598160147667,
    -0.060491856187582016,
    0.10204022377729416,
)


def _p9(dot):
    z = jnp.maximum(0.5 - 0.5 * dot, 1e-30)
    s = z * jax.lax.rsqrt(z)
    g = _G_COEFFS[-1]
    for c in _G_COEFFS[-2::-1]:
        g = c + z * g
    p = 1.0 - s * g
    p2 = p * p
    p4 = p2 * p2
    p8 = p4 * p4
    return p8 * p


def _yoso_block(q_ref, k_ref, v_ref, mrow_ref, o_ref):
    q = q_ref[0]
    k = k_ref[0]
    v = v_ref[0]
    dot = jax.lax.dot_general(
        q, k, (((1,), (1,)), ((), ())), preferred_element_type=jnp.float32)
    p9 = _p9(dot)
    x = jax.lax.dot_general(
        p9, v, (((1,), (0,)), ((), ())), preferred_element_type=jnp.float32)
    x = x * mrow_ref[0]
    norm = jnp.sqrt(jnp.sum(x * x, axis=-1, keepdims=True))
    o_ref[0] = x / (norm + 1e-6)


def kernel(Q, K, V, mask):
    B, H, S, D = Q.shape
    BH = B * H
    mask = mask.astype(Q.dtype)
    Qf = Q.reshape(BH, S, D)
    Kf = K.reshape(BH, S, D)
    Vf = (V * mask[:, None, :, None]).reshape(BH, S, D)
    mrow = mask.reshape(B, S, 1)

    grid = (BH, S // BQ)
    out = pl.pallas_call(
        _yoso_block,
        grid=grid,
        in_specs=[
            pl.BlockSpec((1, BQ, D), lambda h, i: (h, i, 0)),
            pl.BlockSpec((1, S, D), lambda h, i: (h, 0, 0)),
            pl.BlockSpec((1, S, D), lambda h, i: (h, 0, 0)),
            pl.BlockSpec((1, BQ, 1), lambda h, i: (h // H, i, 0)),
        ],
        out_specs=pl.BlockSpec((1, BQ, D), lambda h, i: (h, i, 0)),
        out_shape=jax.ShapeDtypeStruct((BH, S, D), Q.dtype),
        compiler_params=pltpu.CompilerParams(
            dimension_semantics=("arbitrary", "arbitrary"),
        ),
    )(Qf, Kf, Vf, mrow)
    return out.reshape(B, H, S, D)


# speedup vs baseline: 2.2375x; 1.0114x over previous
import math

import jax
import jax.numpy as jnp
from jax.experimental import pallas as pl
from jax.experimental.pallas import tpu as pltpu

HASH_LEN = 9
BQ = 2048

_G_COEFFS = (
    0.6366177201271057,
    0.1062883660197258,
    0.04400774464011192,
    0.05543598160147667,
    -0.060491856187582016,
    0.10204022377729416,
)


def _p9(dot):
    z = jnp.maximum(0.5 - 0.5 * dot, 1e-30)
    s = z * jax.lax.rsqrt(z)
    g = _G_COEFFS[-1]
    for c in _G_COEFFS[-2::-1]:
        g = c + z * g
    p = 1.0 - s * g
    p2 = p * p
    p4 = p2 * p2
    p8 = p4 * p4
    return p8 * p


def _yoso_block(q_ref, k_ref, v_ref, mrow_ref, o_ref):
    q = q_ref[0]
    k = k_ref[0]
    v = v_ref[0]
    dot = jax.lax.dot_general(
        q, k, (((1,), (1,)), ((), ())), preferred_element_type=jnp.float32)
    p9 = _p9(dot)
    x = jax.lax.dot_general(
        p9, v, (((1,), (0,)), ((), ())), preferred_element_type=jnp.float32)
    x = x * mrow_ref[0]
    norm = jnp.sqrt(jnp.sum(x * x, axis=-1, keepdims=True))
    o_ref[0] = x / (norm + 1e-6)


def kernel(Q, K, V, mask):
    B, H, S, D = Q.shape
    BH = B * H
    mask = mask.astype(Q.dtype)
    Qf = Q.reshape(BH, S, D)
    Kf = K.reshape(BH, S, D)
    Vf = (V * mask[:, None, :, None]).reshape(BH, S, D)
    mrow = mask.reshape(B, S, 1)

    grid = (BH, S // BQ)
    out = pl.pallas_call(
        _yoso_block,
        grid=grid,
        in_specs=[
            pl.BlockSpec((1, BQ, D), lambda h, i: (h, i, 0)),
            pl.BlockSpec((1, S, D), lambda h, i: (h, 0, 0)),
            pl.BlockSpec((1, S, D), lambda h, i: (h, 0, 0)),
            pl.BlockSpec((1, BQ, 1), lambda h, i: (h // H, i, 0)),
        ],
        out_specs=pl.BlockSpec((1, BQ, D), lambda h, i: (h, i, 0)),
        out_shape=jax.ShapeDtypeStruct((BH, S, D), Q.dtype),
        compiler_params=pltpu.CompilerParams(
            dimension_semantics=("arbitrary", "arbitrary"),
        ),
    )(Qf, Kf, Vf, mrow)
    return out.reshape(B, H, S, D)


# speedup vs baseline: 2.2791x; 1.0186x over previous
import math

import jax
import jax.numpy as jnp
from jax.experimental import pallas as pl
from jax.experimental.pallas import tpu as pltpu

HASH_LEN = 9
BQ = 2048

_G_COEFFS = (
    0.6366257667541504,
    0.10566413402557373,
    0.05443424731492996,
    -0.004927858244627714,
    0.0755738839507103,
)


def _p9(neghalf_dot):
    z = jnp.maximum(neghalf_dot + 0.5, 1e-30)
    s = z * jax.lax.rsqrt(z)
    g = _G_COEFFS[-1]
    for c in _G_COEFFS[-2::-1]:
        g = c + z * g
    p = jnp.maximum(1.0 - s * g, 1e-30)
    return jnp.exp(jnp.log(p) * 9.0)


def _yoso_block(q_ref, k_ref, v_ref, mrow_ref, o_ref):
    q = q_ref[0]
    k = k_ref[0]
    v = v_ref[0]
    dot = jax.lax.dot_general(
        q, k, (((1,), (1,)), ((), ())), preferred_element_type=jnp.float32)
    p9 = _p9(dot)
    x = jax.lax.dot_general(
        p9, v, (((1,), (0,)), ((), ())), preferred_element_type=jnp.float32)
    x = x * mrow_ref[0]
    n2 = jnp.sum(x * x, axis=-1, keepdims=True)
    o_ref[0] = x * jax.lax.rsqrt(n2 + 1e-24)


def kernel(Q, K, V, mask):
    B, H, S, D = Q.shape
    BH = B * H
    mask = mask.astype(Q.dtype)
    Qf = (Q * -0.5).reshape(BH, S, D)
    Kf = K.reshape(BH, S, D)
    Vf = (V * mask[:, None, :, None]).reshape(BH, S, D)
    mrow = mask.reshape(B, S, 1)

    grid = (BH, S // BQ)
    out = pl.pallas_call(
        _yoso_block,
        grid=grid,
        in_specs=[
            pl.BlockSpec((1, BQ, D), lambda h, i: (h, i, 0)),
            pl.BlockSpec((1, S, D), lambda h, i: (h, 0, 0)),
            pl.BlockSpec((1, S, D), lambda h, i: (h, 0, 0)),
            pl.BlockSpec((1, BQ, 1), lambda h, i: (h // H, i, 0)),
        ],
        out_specs=pl.BlockSpec((1, BQ, D), lambda h, i: (h, i, 0)),
        out_shape=jax.ShapeDtypeStruct((BH, S, D), Q.dtype),
        compiler_params=pltpu.CompilerParams(
            dimension_semantics=("arbitrary", "arbitrary"),
        ),
    )(Qf, Kf, Vf, mrow)
    return out.reshape(B, H, S, D)


# speedup vs baseline: 2.6176x; 1.1485x over previous
import math

import jax
import jax.numpy as jnp
from jax.experimental import pallas as pl
from jax.experimental.pallas import tpu as pltpu

HASH_LEN = 9
BQ = 2048

_G_COEFFS = (
    0.636598527431488,
    0.10729582607746124,
    0.034997887909412384,
    0.06604475528001785,
)


def _p9(neghalf_dot):
    z = jnp.maximum(neghalf_dot + 0.5, 1e-30)
    s = z * jax.lax.rsqrt(z)
    g = _G_COEFFS[-1]
    for c in _G_COEFFS[-2::-1]:
        g = c + z * g
    p = 1.0 - s * g
    p2 = p * p
    p4 = p2 * p2
    p8 = p4 * p4
    return p8 * p


def _yoso_block(q_ref, k_ref, v_ref, mrow_ref, o_ref):
    q = q_ref[0]
    k = k_ref[0]
    v = v_ref[0]
    dot = jax.lax.dot_general(
        q, k, (((1,), (1,)), ((), ())), preferred_element_type=jnp.float32)
    p9 = _p9(dot)
    x = jax.lax.dot_general(
        p9, v, (((1,), (0,)), ((), ())), preferred_element_type=jnp.float32)
    x = x * mrow_ref[0]
    n2 = jnp.sum(x * x, axis=-1, keepdims=True)
    o_ref[0] = x * jax.lax.rsqrt(n2 + 1e-24)


def kernel(Q, K, V, mask):
    B, H, S, D = Q.shape
    BH = B * H
    mask = mask.astype(Q.dtype)
    Qf = (Q * -0.5).reshape(BH, S, D)
    Kf = K.reshape(BH, S, D)
    Vf = (V * mask[:, None, :, None]).reshape(BH, S, D)
    mrow = mask.reshape(B, S, 1)

    grid = (BH, S // BQ)
    out = pl.pallas_call(
        _yoso_block,
        grid=grid,
        in_specs=[
            pl.BlockSpec((1, BQ, D), lambda h, i: (h, i, 0)),
            pl.BlockSpec((1, S, D), lambda h, i: (h, 0, 0)),
            pl.BlockSpec((1, S, D), lambda h, i: (h, 0, 0)),
            pl.BlockSpec((1, BQ, 1), lambda h, i: (h // H, i, 0)),
        ],
        out_specs=pl.BlockSpec((1, BQ, D), lambda h, i: (h, i, 0)),
        out_shape=jax.ShapeDtypeStruct((BH, S, D), Q.dtype),
        compiler_params=pltpu.CompilerParams(
            dimension_semantics=("arbitrary", "arbitrary"),
        ),
    )(Qf, Kf, Vf, mrow)
    return out.reshape(B, H, S, D)


# speedup vs baseline: 2.8923x; 1.1050x over previous
import math

import jax
import jax.numpy as jnp
from jax.experimental import pallas as pl
from jax.experimental.pallas import tpu as pltpu

HASH_LEN = 9
BQ = 2048

_G_COEFFS = (
    0.636598527431488,
    0.10729582607746124,
    0.034997887909412384,
    0.06604475528001785,
)


def _p9(neghalf_dot):
    z = jnp.maximum(neghalf_dot + 0.5, 1e-30)
    s = z * jax.lax.rsqrt(z)
    g = _G_COEFFS[-1]
    for c in _G_COEFFS[-2::-1]:
        g = c + z * g
    p = 1.0 - s * g
    p2 = p * p
    p4 = p2 * p2
    p8 = p4 * p4
    return p8 * p


def _yoso_block(q_ref, k_ref, v_ref, mk_ref, mq_ref, o_ref):
    q = q_ref[0, 0] * -0.5
    k = k_ref[0, 0]
    v = v_ref[0, 0] * mk_ref[0]
    dot = jax.lax.dot_general(
        q, k, (((1,), (1,)), ((), ())), preferred_element_type=jnp.float32)
    p9 = _p9(dot)
    x = jax.lax.dot_general(
        p9, v, (((1,), (0,)), ((), ())), preferred_element_type=jnp.float32)
    x = x * mq_ref[0]
    n2 = jnp.sum(x * x, axis=-1, keepdims=True)
    o_ref[0, 0] = x * jax.lax.rsqrt(n2 + 1e-24)


def kernel(Q, K, V, mask):
    B, H, S, D = Q.shape
    mask3 = mask.astype(Q.dtype).reshape(B, S, 1)

    grid = (B * H, S // BQ)
    out = pl.pallas_call(
        _yoso_block,
        grid=grid,
        in_specs=[
            pl.BlockSpec((1, 1, BQ, D), lambda g, i: (g // H, g % H, i, 0)),
            pl.BlockSpec((1, 1, S, D), lambda g, i: (g // H, g % H, 0, 0)),
            pl.BlockSpec((1, 1, S, D), lambda g, i: (g // H, g % H, 0, 0)),
            pl.BlockSpec((1, S, 1), lambda g, i: (g // H, 0, 0)),
            pl.BlockSpec((1, BQ, 1), lambda g, i: (g // H, i, 0)),
        ],
        out_specs=pl.BlockSpec((1, 1, BQ, D), lambda g, i: (g // H, g % H, i, 0)),
        out_shape=jax.ShapeDtypeStruct((B, H, S, D), Q.dtype),
        compiler_params=pltpu.CompilerParams(
            dimension_semantics=("arbitrary", "arbitrary"),
        ),
    )(Q, K, V, mask3, mask3)
    return out


# speedup vs baseline: 3.3889x; 1.1717x over previous
import math

import jax
import jax.numpy as jnp
from jax.experimental import pallas as pl
from jax.experimental.pallas import tpu as pltpu

HASH_LEN = 9
BQ = 2048

_G_COEFFS = (
    0.636598527431488,
    0.10729582607746124,
    0.034997887909412384,
    0.06604475528001785,
)


def _p9(neghalf_dot):
    z = jnp.maximum(neghalf_dot + 0.5, 1e-30)
    s = z * jax.lax.rsqrt(z)
    g = _G_COEFFS[-1]
    for c in _G_COEFFS[-2::-1]:
        g = c + z * g
    p = 1.0 - s * g
    p2 = p * p
    p4 = p2 * p2
    p8 = p4 * p4
    return p8 * p


def _yoso_block(qt_ref, kt_ref, vt_ref, mk_ref, mq_ref, o_ref):
    qt = qt_ref[0, 0] * -0.5
    kt = kt_ref[0, 0]
    vt = vt_ref[0, 0] * mk_ref[0]
    dot = jax.lax.dot_general(
        qt, kt, (((0,), (0,)), ((), ())), preferred_element_type=jnp.float32)
    p9 = _p9(dot)
    xt = jax.lax.dot_general(
        vt, p9, (((1,), (1,)), ((), ())), preferred_element_type=jnp.float32)
    xt = xt * mq_ref[0]
    n2 = jnp.sum(xt * xt, axis=0, keepdims=True)
    o_ref[0, 0] = xt * jax.lax.rsqrt(n2 + 1e-24)


def kernel(Q, K, V, mask):
    B, H, S, D = Q.shape
    Qt = jnp.transpose(Q, (0, 1, 3, 2))
    Kt = jnp.transpose(K, (0, 1, 3, 2))
    Vt = jnp.transpose(V, (0, 1, 3, 2))
    mask2 = mask.astype(Q.dtype).reshape(B, 1, S)

    grid = (B * H, S // BQ)
    out_t = pl.pallas_call(
        _yoso_block,
        grid=grid,
        in_specs=[
            pl.BlockSpec((1, 1, D, BQ), lambda g, i: (g // H, g % H, 0, i)),
            pl.BlockSpec((1, 1, D, S), lambda g, i: (g // H, g % H, 0, 0)),
            pl.BlockSpec((1, 1, D, S), lambda g, i: (g // H, g % H, 0, 0)),
            pl.BlockSpec((1, 1, S), lambda g, i: (g // H, 0, 0)),
            pl.BlockSpec((1, 1, BQ), lambda g, i: (g // H, 0, i)),
        ],
        out_specs=pl.BlockSpec((1, 1, D, BQ), lambda g, i: (g // H, g % H, 0, i)),
        out_shape=jax.ShapeDtypeStruct((B, H, D, S), Q.dtype),
        compiler_params=pltpu.CompilerParams(
            dimension_semantics=("arbitrary", "arbitrary"),
        ),
    )(Qt, Kt, Vt, mask2, mask2)
    return jnp.transpose(out_t, (0, 1, 3, 2))
